# baseline (speedup 1.0000x reference)
import functools

import jax
import jax.numpy as jnp
from jax import lax
from jax.experimental import pallas as pl
from jax.experimental.pallas import tpu as pltpu
from jax.experimental.pallas import tpu_sc as plsc

NC, NS = 2, 16
NW = NC * NS
BATCH = 16384
D = 64
BPW = BATCH // NW
CHUNK = 128
NCH = BPW // CHUNK


def _sc_gather(users2, items2, umf, imf, umlp, imlp):
  mesh = plsc.VectorSubcoreMesh(core_axis_name="c", subcore_axis_name="s")
  out_t = [jax.ShapeDtypeStruct((BATCH, D), jnp.float32) for _ in range(4)]

  @functools.partial(
      pl.kernel,
      mesh=mesh,
      out_type=out_t,
      compiler_params=pltpu.CompilerParams(use_tc_tiling_on_sc=False),
      scratch_types=[
          pltpu.VMEM((NCH, CHUNK), jnp.int32),
          pltpu.VMEM((NCH, CHUNK), jnp.int32),
          pltpu.VMEM((BPW, D), jnp.float32),
          pltpu.VMEM((BPW, D), jnp.float32),
          pltpu.SemaphoreType.DMA,
          pltpu.SemaphoreType.DMA,
      ],
  )
  def k(users_hbm, items_hbm, umf_hbm, imf_hbm, umlp_hbm, imlp_hbm,
        o_umf, o_imf, o_umlp, o_imlp, idx_u, idx_i, rows0, rows1,
        sem0, sem1):
    wid = lax.axis_index("s") * NC + lax.axis_index("c")
    base = wid * BPW
    pltpu.sync_copy(users_hbm.at[wid], idx_u)
    pltpu.sync_copy(items_hbm.at[wid], idx_i)

    plan = ((umf_hbm, idx_u, o_umf), (imf_hbm, idx_i, o_imf),
            (umlp_hbm, idx_u, o_umlp), (imlp_hbm, idx_i, o_imlp))
    bufs = (rows0, rows1)
    sems = (sem0, sem1)

    pend = None
    for t, (table, idx, out) in enumerate(plan):
      buf = bufs[t % 2]
      sem = sems[t % 2]
      cps = [
          pltpu.async_copy(table.at[idx.at[j]],
                           buf.at[pl.ds(j * CHUNK, CHUNK)], sem)
          for j in range(NCH)
      ]
      if pend is not None:
        pcps, pbuf, pout = pend
        for cp in pcps:
          cp.wait()
        pltpu.sync_copy(pbuf, pout.at[pl.ds(base, BPW)])
      pend = (cps, buf, out)
    pcps, pbuf, pout = pend
    for cp in pcps:
      cp.wait()
    pltpu.sync_copy(pbuf, pout.at[pl.ds(base, BPW)])

  return k(users2, items2, umf, imf, umlp, imlp)


BLK = 1024


def _tc_body(umf, imf, umlp, imlp, w1, b1r, w2, b2r, wf, bfr, out):
  x = jnp.concatenate([umlp[:], imlp[:]], axis=1)
  h = lax.dot_general(x, w1[:], (((1,), (1,)), ((), ())),
                      preferred_element_type=jnp.float32)
  h = jnp.maximum(h + b1r[:], 0.0)
  h = lax.dot_general(h, w2[:], (((1,), (1,)), ((), ())),
                      preferred_element_type=jnp.float32)
  h = jnp.maximum(h + b2r[:], 0.0)
  z = jnp.concatenate([umf[:] * imf[:], h], axis=1)
  out[:] = jnp.sum(z * wf[:], axis=1, keepdims=True) + bfr[0, 0]


def _tc_mlp(umf_r, imf_r, umlp_r, imlp_r, W1, b1, W2, b2, Wf, bf):
  rows = pl.BlockSpec((BLK, D), lambda i: (i, 0))
  full = lambda a: pl.BlockSpec(a.shape, lambda i: tuple(0 for _ in a.shape))
  return pl.pallas_call(
      _tc_body,
      grid=(BATCH // BLK,),
      in_specs=[rows, rows, rows, rows,
                full(W1), full(b1), full(W2), full(b2), full(Wf), full(bf)],
      out_specs=pl.BlockSpec((BLK, 1), lambda i: (i, 0)),
      out_shape=jax.ShapeDtypeStruct((BATCH, 1), jnp.float32),
  )(umf_r, imf_r, umlp_r, imlp_r, W1, b1, W2, b2, Wf, bf)


def kernel(users, items, user_mf, item_mf, user_mlp, item_mlp,
           W1, b1, W2, b2, Wf, bf):
  u2 = users.reshape(NW, NCH, CHUNK)
  i2 = items.reshape(NW, NCH, CHUNK)
  umf_r, imf_r, umlp_r, imlp_r = _sc_gather(
      u2, i2, user_mf, item_mf, user_mlp, item_mlp)
  return _tc_mlp(umf_r, imf_r, umlp_r, imlp_r,
                 W1, b1.reshape(1, -1), W2, b2.reshape(1, -1),
                 Wf, bf.reshape(1, 1))


# speedup vs baseline: 1.4839x; 1.4839x over previous
import functools

import jax
import jax.numpy as jnp
from jax import lax
from jax.experimental import pallas as pl
from jax.experimental.pallas import tpu as pltpu
from jax.experimental.pallas import tpu_sc as plsc

NC, NS = 2, 16
NW = NC * NS
BATCH = 16384
D = 64
BPW = BATCH // NW
CHUNK = 32
NCH = BPW // CHUNK


def _sc_gather(users3, items3, umf, imf, umlp, imlp):
  mesh = plsc.VectorSubcoreMesh(core_axis_name="c", subcore_axis_name="s")
  out_t = [jax.ShapeDtypeStruct((BATCH, D), jnp.float32) for _ in range(4)]

  @functools.partial(
      pl.kernel,
      mesh=mesh,
      out_type=out_t,
      scratch_types=[
          pltpu.VMEM((CHUNK, D), jnp.float32),
          pltpu.VMEM((CHUNK, D), jnp.float32),
          pltpu.VMEM((2 * BPW,), jnp.int32),
          pltpu.SemaphoreType.DMA,
          pltpu.SemaphoreType.DMA,
      ],
  )
  def k(users_hbm, items_hbm, umf_hbm, imf_hbm, umlp_hbm, imlp_hbm,
        o_umf, o_imf, o_umlp, o_imlp,
        buf0, buf1, idx_vmem, sem0, sem1):
    wid = lax.axis_index("s") * NC + lax.axis_index("c")
    base = wid * BPW
    pltpu.sync_copy(users_hbm.at[wid], idx_vmem.at[pl.ds(0, BPW)])
    pltpu.sync_copy(items_hbm.at[wid], idx_vmem.at[pl.ds(BPW, BPW)])

    bufs = (buf0, buf1)
    sems = (sem0, sem1)

    def do_table(table_hbm, off, out_hbm):
      def fire(chunk, sl):
        for p in range(CHUNK // 16):
          vec = idx_vmem[pl.ds(off + chunk * CHUNK + 16 * p, 16)]
          for e in range(16):
            pltpu.make_async_copy(
                table_hbm.at[vec[e]], bufs[sl].at[16 * p + e],
                sems[sl]).start()

      def drain(chunk, sl):
        for e in range(CHUNK):
          pltpu.make_async_copy(
              table_hbm.at[0], bufs[sl].at[e], sems[sl]).wait()

      def flush(chunk, sl):
        pltpu.sync_copy(bufs[sl],
                        out_hbm.at[pl.ds(base + chunk * CHUNK, CHUNK)])

      fire(0, 0)

      def body(i, carry):
        fire(2 * i + 1, 1)
        drain(2 * i, 0)
        flush(2 * i, 0)

        @pl.when(i < NCH // 2 - 1)
        def _():
          fire(2 * i + 2, 0)

        drain(2 * i + 1, 1)
        flush(2 * i + 1, 1)
        return carry

      lax.fori_loop(0, NCH // 2, body, 0)

    do_table(umf_hbm, 0, o_umf)
    do_table(imf_hbm, BPW, o_imf)
    do_table(umlp_hbm, 0, o_umlp)
    do_table(imlp_hbm, BPW, o_imlp)

  return k(users3, items3, umf, imf, umlp, imlp)


BLK = 1024


def _tc_body(umf, imf, umlp, imlp, w1, b1r, w2, b2r, wf, bfr, out):
  x = jnp.concatenate([umlp[:], imlp[:]], axis=1)
  h = lax.dot_general(x, w1[:], (((1,), (1,)), ((), ())),
                      preferred_element_type=jnp.float32)
  h = jnp.maximum(h + b1r[:], 0.0)
  h = lax.dot_general(h, w2[:], (((1,), (1,)), ((), ())),
                      preferred_element_type=jnp.float32)
  h = jnp.maximum(h + b2r[:], 0.0)
  z = jnp.concatenate([umf[:] * imf[:], h], axis=1)
  out[:] = jnp.sum(z * wf[:], axis=1, keepdims=True) + bfr[0, 0]


def _tc_mlp(umf_r, imf_r, umlp_r, imlp_r, W1, b1, W2, b2, Wf, bf):
  rows = pl.BlockSpec((BLK, D), lambda i: (i, 0))
  full = lambda a: pl.BlockSpec(a.shape, lambda i: tuple(0 for _ in a.shape))
  return pl.pallas_call(
      _tc_body,
      grid=(BATCH // BLK,),
      in_specs=[rows, rows, rows, rows,
                full(W1), full(b1), full(W2), full(b2), full(Wf), full(bf)],
      out_specs=pl.BlockSpec((BLK, 1), lambda i: (i, 0)),
      out_shape=jax.ShapeDtypeStruct((BATCH, 1), jnp.float32),
  )(umf_r, imf_r, umlp_r, imlp_r, W1, b1, W2, b2, Wf, bf)


def kernel(users, items, user_mf, item_mf, user_mlp, item_mlp,
           W1, b1, W2, b2, Wf, bf):
  u3 = users.reshape(NW, BPW)
  i3 = items.reshape(NW, BPW)
  umf_r, imf_r, umlp_r, imlp_r = _sc_gather(
      u3, i3, user_mf, item_mf, user_mlp, item_mlp)
  return _tc_mlp(umf_r, imf_r, umlp_r, imlp_r,
                 W1, b1.reshape(1, -1), W2, b2.reshape(1, -1),
                 Wf, bf.reshape(1, 1))
